# baseline (speedup 1.0000x reference)
import functools

import jax
import jax.numpy as jnp
from jax import lax
from jax.experimental import pallas as pl
from jax.experimental.pallas import tpu as pltpu
from jax.experimental.pallas import tpu_sc as plsc

CHUNK = 128


def kernel(chem_id, dis_id, chem_emb, disease_emb):
    B = chem_id.shape[0]
    D = chem_emb.shape[1]
    info = plsc.get_sparse_core_info()
    NC, NS = info.num_cores, info.num_subcores
    NW = NC * NS
    b_per_w = B // NW
    n_chunks = b_per_w // CHUNK
    mesh = plsc.VectorSubcoreMesh(core_axis_name="c", subcore_axis_name="s")

    cid = chem_id.astype(jnp.int32).reshape(NW, n_chunks, CHUNK)
    did = dis_id.astype(jnp.int32).reshape(NW, n_chunks, CHUNK)

    @functools.partial(
        pl.kernel,
        mesh=mesh,
        compiler_params=pltpu.CompilerParams(use_tc_tiling_on_sc=False),
        out_type=(
            jax.ShapeDtypeStruct((B, D), jnp.float32),
            jax.ShapeDtypeStruct((B, D), jnp.float32),
        ),
        scratch_types=[
            pltpu.VMEM((n_chunks, CHUNK), jnp.int32),
            pltpu.VMEM((n_chunks, CHUNK), jnp.int32),
            pltpu.VMEM((b_per_w, D), jnp.float32),
            pltpu.VMEM((b_per_w, D), jnp.float32),
            pltpu.SemaphoreType.DMA,
            pltpu.SemaphoreType.DMA,
        ],
    )
    def _emb(chem_tab, dis_tab, cid_hbm, did_hbm, chem_out, dis_out,
             cidx, didx, crows, drows, csem, dsem):
        wid = lax.axis_index("s") * NC + lax.axis_index("c")
        base = wid * b_per_w
        pltpu.sync_copy(cid_hbm.at[wid], cidx)
        pltpu.sync_copy(did_hbm.at[wid], didx)
        copies = []
        for j in range(n_chunks):
            copies.append(pltpu.async_copy(
                chem_tab.at[cidx.at[j]],
                crows.at[pl.ds(j * CHUNK, CHUNK)], csem))
            copies.append(pltpu.async_copy(
                dis_tab.at[didx.at[j]],
                drows.at[pl.ds(j * CHUNK, CHUNK)], dsem))
        for c in copies:
            c.wait()
        pltpu.sync_copy(crows, chem_out.at[pl.ds(base, b_per_w)])
        pltpu.sync_copy(drows, dis_out.at[pl.ds(base, b_per_w)])

    return _emb(chem_emb, disease_emb, cid, did)


# speedup vs baseline: 1.6706x; 1.6706x over previous
import functools

import jax
import jax.numpy as jnp
from jax import lax
from jax.experimental import pallas as pl
from jax.experimental.pallas import tpu as pltpu
from jax.experimental.pallas import tpu_sc as plsc


def _gather_rows(table, ids):
    B = ids.shape[0]
    D = table.shape[1]
    info = plsc.get_sparse_core_info()
    NC, NS = info.num_cores, info.num_subcores
    NW = NC * NS
    b_per_w = B // NW
    mesh = plsc.VectorSubcoreMesh(core_axis_name="c", subcore_axis_name="s")
    ids2 = ids.astype(jnp.int32).reshape(NW, b_per_w)

    @functools.partial(
        pl.kernel,
        mesh=mesh,
        out_type=jax.ShapeDtypeStruct((B, D), jnp.float32),
        scratch_types=[
            pltpu.VMEM((b_per_w,), jnp.int32),
            pltpu.VMEM((b_per_w, D), jnp.float32),
            pltpu.SemaphoreType.DMA,
        ],
    )
    def _emb(tab, ids_hbm, out, vidx, rows, sem):
        wid = lax.axis_index("s") * NC + lax.axis_index("c")
        base = wid * b_per_w
        pltpu.sync_copy(ids_hbm.at[wid], vidx)

        @pl.loop(0, b_per_w // 16)
        def _fire(k):
            v = vidx[pl.ds(k * 16, 16)]
            for lane in range(16):
                pltpu.async_copy(tab.at[pl.ds(v[lane], 1)],
                                 rows.at[pl.ds(k * 16 + lane, 1)], sem)

        pltpu.make_async_copy(tab.at[pl.ds(0, b_per_w)], rows, sem).wait()

        pltpu.sync_copy(rows, out.at[pl.ds(base, b_per_w)])

    return _emb(table, ids2)


def kernel(chem_id, dis_id, chem_emb, disease_emb):
    return (_gather_rows(chem_emb, chem_id),
            _gather_rows(disease_emb, dis_id))
